# baseline (speedup 1.0000x reference)
import functools

import jax
import jax.numpy as jnp
from jax import lax
from jax.experimental import pallas as pl
from jax.experimental.pallas import tpu as pltpu
from jax.experimental.pallas import tpu_sc as plsc

N_NODES = 10000
N_EDGES = 320000
D_IN = 128
D_HID = 128
N_CLASSES = 47

W1 = 48

NC = 2
NS = 16
NW = NC * NS
EPW = N_EDGES // NW
G = 125
NCH = EPW // G
IGB = 16
RPT_A = 632
RPT_B = N_NODES - 15 * RPT_A


def _make_edge_agg(width):
    mesh = plsc.VectorSubcoreMesh(core_axis_name="c", subcore_axis_name="s")

    @functools.partial(
        pl.kernel,
        mesh=mesh,
        compiler_params=pltpu.CompilerParams(
            use_tc_tiling_on_sc=False,
            disable_bounds_checks=True,
            disable_semaphore_checks=True,
        ),
        out_type=jax.ShapeDtypeStruct((NC, N_NODES, width), jnp.float32),
        scratch_types=[
            pltpu.VMEM((IGB, G), jnp.int32),
            pltpu.VMEM((IGB, G), jnp.int32),
            pltpu.VMEM((G, width), jnp.float32),
            pltpu.VMEM((G, width), jnp.float32),
            pltpu.VMEM_SHARED((N_NODES, width), jnp.float32),
            pltpu.SemaphoreType.DMA,
            pltpu.SemaphoreType.DMA,
        ],
    )
    def edge_agg(tab_hbm, src_hbm, dst_hbm, zeros_hbm, out_hbm,
                 src_v, dst_v, rows_a, rows_b, acc_sh, sem_a, sem_b):
        c = lax.axis_index("c")
        s = lax.axis_index("s")
        wid = s * NC + c
        ebase = pl.multiple_of(wid * NCH, 8)
        rbase = pl.multiple_of(s * RPT_A, 8)

        @pl.when(s < NS - 1)
        def _():
            pltpu.sync_copy(zeros_hbm, acc_sh.at[pl.ds(rbase, RPT_A)])

        @pl.when(s == NS - 1)
        def _():
            pltpu.sync_copy(zeros_hbm.at[pl.ds(0, RPT_B)],
                            acc_sh.at[pl.ds(rbase, RPT_B)])

        plsc.subcore_barrier()

        for g in range(NCH // IGB):
            pltpu.sync_copy(src_hbm.at[pl.ds(ebase + g * IGB, IGB)], src_v)
            pltpu.sync_copy(dst_hbm.at[pl.ds(ebase + g * IGB, IGB)], dst_v)

            pltpu.async_copy(tab_hbm.at[src_v.at[0]], rows_a, sem_a)

            def pair(i, carry):
                pltpu.async_copy(tab_hbm.at[src_v.at[2 * i + 1]],
                                 rows_b, sem_b)
                pltpu.make_async_copy(tab_hbm.at[src_v.at[0]],
                                      rows_a, sem_a).wait()
                pltpu.sync_copy(rows_a, acc_sh.at[dst_v.at[2 * i]], add=True)

                @pl.when(i < IGB // 2 - 1)
                def _():
                    pltpu.async_copy(tab_hbm.at[src_v.at[2 * i + 2]],
                                     rows_a, sem_a)

                pltpu.make_async_copy(tab_hbm.at[src_v.at[0]],
                                      rows_b, sem_b).wait()
                pltpu.sync_copy(rows_b, acc_sh.at[dst_v.at[2 * i + 1]],
                                add=True)
                return carry

            lax.fori_loop(0, IGB // 2, pair, 0)
        plsc.subcore_barrier()

        @pl.when(s < NS - 1)
        def _():
            pltpu.sync_copy(acc_sh.at[pl.ds(rbase, RPT_A)],
                            out_hbm.at[c, pl.ds(rbase, RPT_A)])

        @pl.when(s == NS - 1)
        def _():
            pltpu.sync_copy(acc_sh.at[pl.ds(rbase, RPT_B)],
                            out_hbm.at[c, pl.ds(rbase, RPT_B)])

    return edge_agg


_edge_agg_l1 = _make_edge_agg(W1)

WS = 72
NCH2 = (N_EDGES // NS) // G


def _make_edge_agg_split():
    mesh = plsc.VectorSubcoreMesh(core_axis_name="c", subcore_axis_name="s")

    @functools.partial(
        pl.kernel,
        mesh=mesh,
        compiler_params=pltpu.CompilerParams(
            use_tc_tiling_on_sc=False,
            disable_bounds_checks=True,
            disable_semaphore_checks=True,
        ),
        out_type=[
            jax.ShapeDtypeStruct((N_NODES, WS), jnp.float32),
            jax.ShapeDtypeStruct((N_NODES, WS), jnp.float32),
        ],
        scratch_types=[
            pltpu.VMEM((IGB, G), jnp.int32),
            pltpu.VMEM((IGB, G), jnp.int32),
            pltpu.VMEM((G, WS), jnp.float32),
            pltpu.VMEM((G, WS), jnp.float32),
            pltpu.VMEM_SHARED((N_NODES, WS), jnp.float32),
            pltpu.SemaphoreType.DMA,
            pltpu.SemaphoreType.DMA,
        ],
    )
    def edge_agg(taba_hbm, tabb_hbm, src_hbm, dst_hbm, zeros_hbm,
                 outa_hbm, outb_hbm,
                 src_v, dst_v, rows_a, rows_b, acc_sh, sem_a, sem_b):
        c = lax.axis_index("c")
        s = lax.axis_index("s")
        ebase = pl.multiple_of(s * NCH2, 8)
        rbase = pl.multiple_of(s * RPT_A, 8)

        @pl.when(s < NS - 1)
        def _():
            pltpu.sync_copy(zeros_hbm, acc_sh.at[pl.ds(rbase, RPT_A)])

        @pl.when(s == NS - 1)
        def _():
            pltpu.sync_copy(zeros_hbm.at[pl.ds(0, RPT_B)],
                            acc_sh.at[pl.ds(rbase, RPT_B)])

        plsc.subcore_barrier()

        def run(tab_hbm):
            for g in range(NCH2 // IGB):
                pltpu.sync_copy(src_hbm.at[pl.ds(ebase + g * IGB, IGB)],
                                src_v)
                pltpu.sync_copy(dst_hbm.at[pl.ds(ebase + g * IGB, IGB)],
                                dst_v)

                pltpu.async_copy(tab_hbm.at[src_v.at[0]], rows_a, sem_a)

                def pair(i, carry):
                    pltpu.async_copy(tab_hbm.at[src_v.at[2 * i + 1]],
                                     rows_b, sem_b)
                    pltpu.make_async_copy(tab_hbm.at[src_v.at[0]],
                                          rows_a, sem_a).wait()
                    pltpu.sync_copy(rows_a, acc_sh.at[dst_v.at[2 * i]],
                                    add=True)

                    @pl.when(i < IGB // 2 - 1)
                    def _():
                        pltpu.async_copy(tab_hbm.at[src_v.at[2 * i + 2]],
                                         rows_a, sem_a)

                    pltpu.make_async_copy(tab_hbm.at[src_v.at[0]],
                                          rows_b, sem_b).wait()
                    pltpu.sync_copy(rows_b, acc_sh.at[dst_v.at[2 * i + 1]],
                                    add=True)
                    return carry

                lax.fori_loop(0, IGB // 2, pair, 0)

        @pl.when(c == 0)
        def _():
            run(taba_hbm)

        @pl.when(c == 1)
        def _():
            run(tabb_hbm)

        plsc.subcore_barrier()

        def writeback(out_hbm):
            @pl.when(s < NS - 1)
            def _():
                pltpu.sync_copy(acc_sh.at[pl.ds(rbase, RPT_A)],
                                out_hbm.at[pl.ds(rbase, RPT_A)])

            @pl.when(s == NS - 1)
            def _():
                pltpu.sync_copy(acc_sh.at[pl.ds(rbase, RPT_B)],
                                out_hbm.at[pl.ds(rbase, RPT_B)])

        @pl.when(c == 0)
        def _():
            writeback(outa_hbm)

        @pl.when(c == 1)
        def _():
            writeback(outb_hbm)

    return edge_agg


_edge_agg_l0s = _make_edge_agg_split()


def _tc_main_body(x_ref, pa_ref, pb_ref, ws0_ref, wn0_ref, b0_ref,
                  wn1_ref, ws1_ref, b1_ref, proj_ref, self_ref, inv_ref):
    deg = pa_ref[:, 64:65]
    inv = 1.0 / jnp.maximum(deg, 1.0)
    h_neigh = jnp.concatenate([pa_ref[:, :64], pb_ref[:, :64]], axis=1) * inv
    h1 = x_ref[...] @ ws0_ref[...] + h_neigh @ wn0_ref[...] + b0_ref[...]
    h1 = jnp.maximum(h1, 0.0)
    proj_ref[...] = h1 @ wn1_ref[...]
    self_ref[...] = h1 @ ws1_ref[...] + b1_ref[...]
    inv_ref[...] = inv


def _tc_epilogue_body(self_ref, a0_ref, a1_ref, inv_ref, out_ref):
    agg = a0_ref[0] + a1_ref[0]
    out_ref[...] = self_ref[...] + agg * inv_ref[...]


def kernel(x, edge_index, W_self_0, W_neigh_0, b_0, W_self_1, W_neigh_1, b_1):
    src = edge_index[0].astype(jnp.int32).reshape(N_EDGES // G, G)
    dst = edge_index[1].astype(jnp.int32).reshape(N_EDGES // G, G)

    pad7 = jnp.zeros((N_NODES, 7), jnp.float32)
    taba = jnp.concatenate([x[:, :64], pad7[:, :1] + 1.0, pad7], axis=1)
    tabb = jnp.concatenate([x[:, 64:], pad7, pad7[:, :1]], axis=1)
    zeros0 = jnp.zeros((RPT_A, WS), jnp.float32)
    zeros1 = jnp.zeros((RPT_A, W1), jnp.float32)

    pa, pb = _edge_agg_l0s(taba, tabb, src, dst, zeros0)

    wn1p = jnp.zeros((D_HID, W1), jnp.float32).at[:, :N_CLASSES].set(W_neigh_1)
    ws1p = jnp.zeros((D_HID, W1), jnp.float32).at[:, :N_CLASSES].set(W_self_1)
    b1p = jnp.zeros((1, W1), jnp.float32).at[0, :N_CLASSES].set(b_1)

    BR = 1000
    grid = (N_NODES // BR,)
    proj, self1, inv = pl.pallas_call(
        _tc_main_body,
        grid=grid,
        in_specs=[
            pl.BlockSpec((BR, D_IN), lambda i: (i, 0)),
            pl.BlockSpec((BR, WS), lambda i: (i, 0)),
            pl.BlockSpec((BR, WS), lambda i: (i, 0)),
            pl.BlockSpec((D_IN, D_HID), lambda i: (0, 0)),
            pl.BlockSpec((D_IN, D_HID), lambda i: (0, 0)),
            pl.BlockSpec((1, D_HID), lambda i: (0, 0)),
            pl.BlockSpec((D_HID, W1), lambda i: (0, 0)),
            pl.BlockSpec((D_HID, W1), lambda i: (0, 0)),
            pl.BlockSpec((1, W1), lambda i: (0, 0)),
        ],
        out_specs=[
            pl.BlockSpec((BR, W1), lambda i: (i, 0)),
            pl.BlockSpec((BR, W1), lambda i: (i, 0)),
            pl.BlockSpec((BR, 1), lambda i: (i, 0)),
        ],
        out_shape=[
            jax.ShapeDtypeStruct((N_NODES, W1), jnp.float32),
            jax.ShapeDtypeStruct((N_NODES, W1), jnp.float32),
            jax.ShapeDtypeStruct((N_NODES, 1), jnp.float32),
        ],
    )(x, pa, pb, W_self_0, W_neigh_0, b_0.reshape(1, D_HID),
      wn1p, ws1p, b1p)

    part1 = _edge_agg_l1(proj, src, dst, zeros1)

    out48 = pl.pallas_call(
        _tc_epilogue_body,
        grid=grid,
        in_specs=[
            pl.BlockSpec((BR, W1), lambda i: (i, 0)),
            pl.BlockSpec((1, BR, W1), lambda i: (0, i, 0)),
            pl.BlockSpec((1, BR, W1), lambda i: (1, i, 0)),
            pl.BlockSpec((BR, 1), lambda i: (i, 0)),
        ],
        out_specs=pl.BlockSpec((BR, W1), lambda i: (i, 0)),
        out_shape=jax.ShapeDtypeStruct((N_NODES, W1), jnp.float32),
    )(self1, part1, part1, inv)

    return out48[:, :N_CLASSES]
